# baseline (speedup 1.0000x reference)
import functools

import jax
import jax.numpy as jnp
from jax import lax
from jax.experimental import pallas as pl
from jax.experimental.pallas import tpu as pltpu
from jax.experimental.pallas import tpu_sc as plsc

N = 10000
E = 320000
D = 128
C = 64

NC = 2
NS = 16
NW = NC * NS

CH = 128
EPT = E // NW
CPT = 80
CPTP = CPT + 2
IPT = CPTP * CH
N_PAD = 10240
RPT = N_PAD // NS


@functools.cache
def _sc_kernels():
    mesh = plsc.VectorSubcoreMesh(core_axis_name="c", subcore_axis_name="s")
    deg = functools.partial(
        pl.kernel,
        mesh=mesh,
        out_type=jax.ShapeDtypeStruct((2, N_PAD, D), jnp.float32),
        scratch_types=[
            pltpu.VMEM((IPT,), jnp.int32),
            pltpu.VMEM((CH,), jnp.int32),
            pltpu.VMEM((CH, D), jnp.float32),
            pltpu.VMEM_SHARED((N_PAD, D), jnp.float32),
        ],
    )(_deg_body)
    hop = functools.partial(
        pl.kernel,
        mesh=mesh,
        out_type=jax.ShapeDtypeStruct((2, N_PAD, D), jnp.float32),
        scratch_types=[
            pltpu.VMEM((IPT,), jnp.int32),
            pltpu.VMEM((IPT,), jnp.int32),
            pltpu.VMEM((CH,), jnp.int32),
            pltpu.VMEM((CH,), jnp.int32),
            pltpu.VMEM((CH, D), jnp.float32),
            pltpu.VMEM_SHARED((N_PAD, D), jnp.float32),
            pltpu.SemaphoreType.DMA,
        ],
    )(_hop_body)
    return deg, hop


def _zero_and_init_acc(buf, acc, s):
    def _zrow(i, carry):
        for j in range(D // 16):
            buf[i, pl.ds(j * 16, 16)] = jnp.zeros((16,), jnp.float32)
        return carry

    lax.fori_loop(0, CH, _zrow, 0)
    for k in range(RPT // CH):
        pltpu.sync_copy(buf, acc.at[pl.ds(s * RPT + k * CH, CH)])
    plsc.subcore_barrier()


def _tec_copy_idx(src_all, dst, j):
    base = j * CH
    for k in range(CH // 16):
        dst[pl.ds(k * 16, 16)] = src_all[pl.ds(base + k * 16, 16)]


def _deg_body(col_hbm, out_hbm, call_v, cidx, buf, acc):
    c = lax.axis_index("c")
    s = lax.axis_index("s")
    wid = s * NC + c

    _zero_and_init_acc(buf, acc, s)

    def _orow(i, carry):
        for j in range(D // 16):
            buf[i, pl.ds(j * 16, 16)] = jnp.ones((16,), jnp.float32)
        return carry

    lax.fori_loop(0, CH, _orow, 0)

    pltpu.sync_copy(col_hbm.at[pl.ds(wid * IPT, IPT)], call_v)

    def _body(j, carry):
        _tec_copy_idx(call_v, cidx, j)
        pltpu.sync_copy(buf, acc.at[cidx], add=True)
        return carry

    lax.fori_loop(0, CPT, _body, 0)
    plsc.subcore_barrier()
    pltpu.sync_copy(acc.at[pl.ds(s * RPT, RPT)], out_hbm.at[c, pl.ds(s * RPT, RPT)])


def _hop_body(g_hbm, row_hbm, col_hbm, out_hbm, rall_v, call_v, ridx, cidx,
              msg, acc, sg):
    c = lax.axis_index("c")
    s = lax.axis_index("s")
    wid = s * NC + c

    _zero_and_init_acc(msg, acc, s)

    pltpu.sync_copy(row_hbm.at[pl.ds(wid * IPT, IPT)], rall_v)
    pltpu.sync_copy(col_hbm.at[pl.ds(wid * IPT, IPT)], call_v)

    _tec_copy_idx(rall_v, ridx, 0)
    _tec_copy_idx(call_v, cidx, 0)
    pltpu.async_copy(g_hbm.at[ridx], msg, sg)

    def _body(j, carry):
        pltpu.make_async_copy(g_hbm.at[ridx], msg, sg).wait()
        pltpu.sync_copy(msg, acc.at[cidx], add=True)
        _tec_copy_idx(rall_v, ridx, j + 1)
        _tec_copy_idx(call_v, cidx, j + 1)
        pltpu.async_copy(g_hbm.at[ridx], msg, sg)
        return carry

    lax.fori_loop(0, CPT, _body, 0)
    pltpu.make_async_copy(g_hbm.at[ridx], msg, sg).wait()

    plsc.subcore_barrier()
    pltpu.sync_copy(acc.at[pl.ds(s * RPT, RPT)], out_hbm.at[c, pl.ds(s * RPT, RPT)])


_RB = 1024


def _deg_col(dp_blk):
    return dp_blk[0, :, 0:1] + dp_blk[1, :, 0:1] + 1.0


def _scale1_body(dp, x, o):
    o[...] = x[...] * lax.rsqrt(_deg_col(dp[...]))


def _scale1(dp, x):
    grid = N_PAD // _RB
    return pl.pallas_call(
        _scale1_body,
        grid=(grid,),
        in_specs=[
            pl.BlockSpec((2, _RB, D), lambda i: (0, i, 0)),
            pl.BlockSpec((_RB, D), lambda i: (i, 0)),
        ],
        out_specs=pl.BlockSpec((_RB, D), lambda i: (i, 0)),
        out_shape=jax.ShapeDtypeStruct((N_PAD, D), jnp.float32),
    )(dp, x)


def _scale2_body(dp, pp, g, o):
    tot = pp[0] + pp[1] + g[...]
    o[...] = tot / _deg_col(dp[...])


def _scale2(dp, pp, g):
    grid = N_PAD // _RB
    return pl.pallas_call(
        _scale2_body,
        grid=(grid,),
        in_specs=[
            pl.BlockSpec((2, _RB, D), lambda i: (0, i, 0)),
            pl.BlockSpec((2, _RB, D), lambda i: (0, i, 0)),
            pl.BlockSpec((_RB, D), lambda i: (i, 0)),
        ],
        out_specs=pl.BlockSpec((_RB, D), lambda i: (i, 0)),
        out_shape=jax.ShapeDtypeStruct((N_PAD, D), jnp.float32),
    )(dp, pp, g)


def _final_body(dp, pp, g, wt, b, o):
    h2 = (pp[0] + pp[1] + g[...]) * lax.rsqrt(_deg_col(dp[...]))
    logits = jnp.dot(h2, wt[...], preferred_element_type=jnp.float32) + b[...]
    m = jnp.max(logits, axis=1, keepdims=True)
    z = logits - m
    lse = jnp.log(jnp.sum(jnp.exp(z), axis=1, keepdims=True))
    o[...] = z - lse


def _final(dp, pp, g, wt, b2):
    grid = N_PAD // _RB
    return pl.pallas_call(
        _final_body,
        grid=(grid,),
        in_specs=[
            pl.BlockSpec((2, _RB, D), lambda i: (0, i, 0)),
            pl.BlockSpec((2, _RB, D), lambda i: (0, i, 0)),
            pl.BlockSpec((_RB, D), lambda i: (i, 0)),
            pl.BlockSpec((D, C), lambda i: (0, 0)),
            pl.BlockSpec((1, C), lambda i: (0, 0)),
        ],
        out_specs=pl.BlockSpec((_RB, C), lambda i: (i, 0)),
        out_shape=jax.ShapeDtypeStruct((N_PAD, C), jnp.float32),
    )(dp, pp, g, wt, b2)


@jax.jit
def kernel(x, edge_index, W, b):
    row = edge_index[0].astype(jnp.int32).reshape(NW, EPT)
    col = edge_index[1].astype(jnp.int32).reshape(NW, EPT)
    padw = CPTP * CH - EPT
    row = jnp.pad(row, ((0, 0), (0, padw)), constant_values=N).reshape(-1)
    col = jnp.pad(col, ((0, 0), (0, padw)), constant_values=N).reshape(-1)
    x_pad = jnp.pad(x, ((0, N_PAD - N), (0, 0)))

    deg_k, hop_k = _sc_kernels()
    dp = deg_k(col)
    g1 = _scale1(dp, x_pad)
    pp1 = hop_k(g1, row, col)
    g2 = _scale2(dp, pp1, g1)
    pp2 = hop_k(g2, row, col)
    out = _final(dp, pp2, g2, W.T, b.reshape(1, C))
    return out[:N]


# speedup vs baseline: 1.4747x; 1.4747x over previous
import functools

import jax
import jax.numpy as jnp
from jax import lax
from jax.experimental import pallas as pl
from jax.experimental.pallas import tpu as pltpu
from jax.experimental.pallas import tpu_sc as plsc

N = 10000
E = 320000
D = 128
C = 64

NC = 2
NS = 16
NW = NC * NS

CH = 128
CPT = -(-E // (NW * CH))
E_PAD = NW * CPT * CH
N_PAD = 10240
RPT = N_PAD // NS


@functools.cache
def _sc_kernels():
    mesh = plsc.VectorSubcoreMesh(core_axis_name="c", subcore_axis_name="s")
    deg = functools.partial(
        pl.kernel,
        mesh=mesh,
        out_type=jax.ShapeDtypeStruct((2, N_PAD, D), jnp.float32),
        scratch_types=[
            pltpu.VMEM((CH,), jnp.int32),
            pltpu.VMEM((CH, D), jnp.float32),
            pltpu.VMEM_SHARED((N_PAD, D), jnp.float32),
        ],
    )(_deg_body)
    hop = functools.partial(
        pl.kernel,
        mesh=mesh,
        out_type=jax.ShapeDtypeStruct((2, N_PAD, D), jnp.float32),
        scratch_types=[
            pltpu.VMEM((CH,), jnp.int32),
            pltpu.VMEM((CH,), jnp.int32),
            pltpu.VMEM((CH, D), jnp.float32),
            pltpu.VMEM_SHARED((N_PAD, D), jnp.float32),
            pltpu.SemaphoreType.DMA,
        ],
    )(_hop_body)
    return deg, hop


def _deg_body(col_hbm, out_hbm, cidx, buf, acc):
    c = lax.axis_index("c")
    s = lax.axis_index("s")
    wid = s * NC + c

    def _zrow(i, carry):
        for j in range(D // 16):
            buf[i, pl.ds(j * 16, 16)] = jnp.zeros((16,), jnp.float32)
        return carry

    lax.fori_loop(0, CH, _zrow, 0)
    for k in range(RPT // CH):
        pltpu.sync_copy(buf, acc.at[pl.ds(s * RPT + k * CH, CH)])
    plsc.subcore_barrier()

    def _orow(i, carry):
        for j in range(D // 16):
            buf[i, pl.ds(j * 16, 16)] = jnp.ones((16,), jnp.float32)
        return carry

    lax.fori_loop(0, CH, _orow, 0)

    def _body(j, carry):
        off = (wid * CPT + j) * CH
        pltpu.sync_copy(col_hbm.at[pl.ds(off, CH)], cidx)
        pltpu.sync_copy(buf, acc.at[cidx], add=True)
        return carry

    lax.fori_loop(0, CPT, _body, 0)
    plsc.subcore_barrier()
    pltpu.sync_copy(acc.at[pl.ds(s * RPT, RPT)], out_hbm.at[c, pl.ds(s * RPT, RPT)])


def _hop_body(g_hbm, row_hbm, col_hbm, out_hbm, ridx, cidx, msg, acc, sem):
    c = lax.axis_index("c")
    s = lax.axis_index("s")
    wid = s * NC + c

    def _zrow(i, carry):
        for j in range(D // 16):
            msg[i, pl.ds(j * 16, 16)] = jnp.zeros((16,), jnp.float32)
        return carry

    lax.fori_loop(0, CH, _zrow, 0)
    for k in range(RPT // CH):
        pltpu.sync_copy(msg, acc.at[pl.ds(s * RPT + k * CH, CH)])
    plsc.subcore_barrier()

    def _body(j, carry):
        off = (wid * CPT + j) * CH
        pltpu.sync_copy(row_hbm.at[pl.ds(off, CH)], ridx)
        pltpu.sync_copy(col_hbm.at[pl.ds(off, CH)], cidx)
        pltpu.async_copy(g_hbm.at[ridx], msg, sem).wait()
        pltpu.sync_copy(msg, acc.at[cidx], add=True)
        return carry

    lax.fori_loop(0, CPT, _body, 0)
    plsc.subcore_barrier()
    pltpu.sync_copy(acc.at[pl.ds(s * RPT, RPT)], out_hbm.at[c, pl.ds(s * RPT, RPT)])


_RB = 1024


def _deg_col(dp_blk):
    return dp_blk[0, :, 0:1] + dp_blk[1, :, 0:1] + 1.0


def _scale1_body(dp, x, o):
    o[...] = x[...] * lax.rsqrt(_deg_col(dp[...]))


def _scale1(dp, x):
    grid = N_PAD // _RB
    return pl.pallas_call(
        _scale1_body,
        grid=(grid,),
        in_specs=[
            pl.BlockSpec((2, _RB, D), lambda i: (0, i, 0)),
            pl.BlockSpec((_RB, D), lambda i: (i, 0)),
        ],
        out_specs=pl.BlockSpec((_RB, D), lambda i: (i, 0)),
        out_shape=jax.ShapeDtypeStruct((N_PAD, D), jnp.float32),
    )(dp, x)


def _scale2_body(dp, pp, g, o):
    tot = pp[0] + pp[1] + g[...]
    o[...] = tot / _deg_col(dp[...])


def _scale2(dp, pp, g):
    grid = N_PAD // _RB
    return pl.pallas_call(
        _scale2_body,
        grid=(grid,),
        in_specs=[
            pl.BlockSpec((2, _RB, D), lambda i: (0, i, 0)),
            pl.BlockSpec((2, _RB, D), lambda i: (0, i, 0)),
            pl.BlockSpec((_RB, D), lambda i: (i, 0)),
        ],
        out_specs=pl.BlockSpec((_RB, D), lambda i: (i, 0)),
        out_shape=jax.ShapeDtypeStruct((N_PAD, D), jnp.float32),
    )(dp, pp, g)


def _final_body(dp, pp, g, wt, b, o):
    h2 = (pp[0] + pp[1] + g[...]) * lax.rsqrt(_deg_col(dp[...]))
    logits = jnp.dot(h2, wt[...], preferred_element_type=jnp.float32) + b[...]
    m = jnp.max(logits, axis=1, keepdims=True)
    z = logits - m
    lse = jnp.log(jnp.sum(jnp.exp(z), axis=1, keepdims=True))
    o[...] = z - lse


def _final(dp, pp, g, wt, b2):
    grid = N_PAD // _RB
    return pl.pallas_call(
        _final_body,
        grid=(grid,),
        in_specs=[
            pl.BlockSpec((2, _RB, D), lambda i: (0, i, 0)),
            pl.BlockSpec((2, _RB, D), lambda i: (0, i, 0)),
            pl.BlockSpec((_RB, D), lambda i: (i, 0)),
            pl.BlockSpec((D, C), lambda i: (0, 0)),
            pl.BlockSpec((1, C), lambda i: (0, 0)),
        ],
        out_specs=pl.BlockSpec((_RB, C), lambda i: (i, 0)),
        out_shape=jax.ShapeDtypeStruct((N_PAD, C), jnp.float32),
    )(dp, pp, g, wt, b2)


@jax.jit
def kernel(x, edge_index, W, b):
    row = edge_index[0].astype(jnp.int32)
    col = edge_index[1].astype(jnp.int32)
    pad = jnp.full((E_PAD - E,), N, jnp.int32)
    row = jnp.concatenate([row, pad])
    col = jnp.concatenate([col, pad])
    x_pad = jnp.pad(x, ((0, N_PAD - N), (0, 0)))

    deg_k, hop_k = _sc_kernels()
    dp = deg_k(col)
    g1 = _scale1(dp, x_pad)
    pp1 = hop_k(g1, row, col)
    g2 = _scale2(dp, pp1, g1)
    pp2 = hop_k(g2, row, col)
    out = _final(dp, pp2, g2, W.T, b.reshape(1, C))
    return out[:N]
